# Initial kernel scaffold; baseline (speedup 1.0000x reference)
import jax
import jax.numpy as jnp
from jax.experimental import pallas as pl


def kernel(x, edge_index, W1, b1, S1w, S1b, W2, b2, S2w, S2b, Cw, Cb):
    raise NotImplementedError("write your pallas kernel here")


# speedup vs baseline: 17.8370x; 17.8370x over previous
import dataclasses
import functools

import jax
import jax.numpy as jnp
from jax import lax
from jax.experimental import pallas as pl
from jax.experimental.pallas import tpu as pltpu
from jax.experimental.pallas import tpu_sc as plsc

N = 10000
E = 320000
D_IN = 128
H = 32
C = 16

NC = 2
NS = 16
L = 16
NW = NC * NS
BLK = 128
NBLK = 79
PERW = NBLK * BLK
EPAD = NW * PERW
NP = 10240
DEG_ROWS = NP // L
DR_SUB = DEG_ROWS // NS
NP_SUB = NP // NS

_mesh = plsc.VectorSubcoreMesh(core_axis_name="c", subcore_axis_name="s")

_sc_params = pltpu.CompilerParams()
if "needs_layout_passes" in pltpu.CompilerParams.__dataclass_fields__:
    _sc_params = dataclasses.replace(_sc_params, needs_layout_passes=False)
if "use_tc_tiling_on_sc" in pltpu.CompilerParams.__dataclass_fields__:
    _sc_params = dataclasses.replace(_sc_params, use_tc_tiling_on_sc=False)


def _deg_body(dst_hbm, iota_hbm, z_hbm, out_hbm, didx, hist, ibuf, deg_sh):
    c = lax.axis_index("c")
    s = lax.axis_index("s")
    w = s * NC + c
    pltpu.sync_copy(z_hbm, hist)
    pltpu.sync_copy(z_hbm.at[pl.ds(s * DR_SUB, DR_SUB)],
                    deg_sh.at[pl.ds(s * DR_SUB, DR_SUB)])
    plsc.subcore_barrier()

    ones = jnp.ones((L,), jnp.float32)
    base = w * PERW

    @pl.loop(0, NBLK)
    def _blocks(i):
        pltpu.sync_copy(dst_hbm.at[pl.ds(base + i * BLK, BLK)], didx)

        @pl.loop(0, BLK // L)
        def _vecs(j):
            idx = didx[pl.ds(j * L, L)]
            row = jnp.right_shift(idx, 4)
            lane = jnp.bitwise_and(idx, 15)
            plsc.addupdate_scatter(hist, [row, lane], ones)

    @pl.loop(0, DEG_ROWS // BLK)
    def _merge(k):
        pltpu.sync_copy(iota_hbm.at[pl.ds(k * BLK, BLK)], ibuf)
        pltpu.sync_copy(hist.at[pl.ds(k * BLK, BLK)], deg_sh.at[ibuf], add=True)

    plsc.subcore_barrier()
    pltpu.sync_copy(deg_sh.at[pl.ds(s * DR_SUB, DR_SUB)],
                    out_hbm.at[c, pl.ds(s * DR_SUB, DR_SUB)])


_deg_pass = pl.kernel(
    _deg_body,
    out_type=jax.ShapeDtypeStruct((NC, DEG_ROWS, L), jnp.float32),
    mesh=_mesh,
    scratch_types=[
        pltpu.VMEM((BLK,), jnp.int32),
        pltpu.VMEM((DEG_ROWS, L), jnp.float32),
        pltpu.VMEM((BLK,), jnp.int32),
        pltpu.VMEM_SHARED((DEG_ROWS, L), jnp.float32),
    ],
    compiler_params=_sc_params,
)


def _edge_body(g_hbm, src_hbm, dst_hbm, z_hbm, out_hbm,
               sidx, didx, rows, acc_sh, sem):
    c = lax.axis_index("c")
    s = lax.axis_index("s")
    w = s * NC + c
    pltpu.sync_copy(z_hbm.at[pl.ds(s * NP_SUB, NP_SUB)],
                    acc_sh.at[pl.ds(s * NP_SUB, NP_SUB)])
    plsc.subcore_barrier()

    base = w * PERW

    @pl.loop(0, NBLK)
    def _blocks(i):
        off = base + i * BLK
        pltpu.sync_copy(src_hbm.at[pl.ds(off, BLK)], sidx)
        pltpu.async_copy(g_hbm.at[sidx], rows, sem).wait()
        pltpu.sync_copy(dst_hbm.at[pl.ds(off, BLK)], didx)
        pltpu.sync_copy(rows, acc_sh.at[didx], add=True)

    plsc.subcore_barrier()
    pltpu.sync_copy(acc_sh.at[pl.ds(s * NP_SUB, NP_SUB)],
                    out_hbm.at[c, pl.ds(s * NP_SUB, NP_SUB)])


_edge_pass = pl.kernel(
    _edge_body,
    out_type=jax.ShapeDtypeStruct((NC, NP, H), jnp.float32),
    mesh=_mesh,
    scratch_types=[
        pltpu.VMEM((BLK,), jnp.int32),
        pltpu.VMEM((BLK,), jnp.int32),
        pltpu.VMEM((BLK, H), jnp.float32),
        pltpu.VMEM_SHARED((NP, H), jnp.float32),
        pltpu.SemaphoreType.DMA,
    ],
    compiler_params=_sc_params,
)


def _mm_body(x_ref, w_ref, o_ref):
    o_ref[...] = jnp.dot(x_ref[...], w_ref[...],
                         preferred_element_type=jnp.float32)


def _tc_mm(x, w):
    return pl.pallas_call(
        _mm_body,
        out_shape=jax.ShapeDtypeStruct((x.shape[0], w.shape[1]), jnp.float32),
    )(x, w)


def _scale_body(u_ref, d0_ref, d1_ref, g1_ref, dinv_ref):
    deg = d0_ref[...] + d1_ref[...] + 1.0
    dinv = lax.rsqrt(deg)
    dinv_ref[...] = dinv
    g1_ref[...] = u_ref[:, :H] * dinv[:N, :]


def _tc_scale(u, d0, d1):
    return pl.pallas_call(
        _scale_body,
        out_shape=(jax.ShapeDtypeStruct((N, H), jnp.float32),
                   jax.ShapeDtypeStruct((NP, 1), jnp.float32)),
    )(u, d0, d1)


def _layer_body(a0_ref, a1_ref, g1_ref, dinv_ref, u_ref, b_ref, w2_ref,
                g2_ref, v2_ref):
    dv = dinv_ref[:N, :]
    h1 = jnp.maximum(
        (a0_ref[:N, :] + a1_ref[:N, :] + g1_ref[...]) * dv
        + u_ref[:, H:] + b_ref[...], 0.0)
    v = jnp.dot(h1, w2_ref[...], preferred_element_type=jnp.float32)
    g2_ref[...] = v[:, :H] * dv
    v2_ref[...] = v[:, H:]


def _tc_layer(a0, a1, g1, dinv, u, b, w2):
    return pl.pallas_call(
        _layer_body,
        out_shape=(jax.ShapeDtypeStruct((N, H), jnp.float32),
                   jax.ShapeDtypeStruct((N, H), jnp.float32)),
    )(a0, a1, g1, dinv, u, b, w2)


def _final_body(a0_ref, a1_ref, g2_ref, dinv_ref, v2_ref, b_ref, cw_ref,
                cb_ref, o_ref):
    dv = dinv_ref[:N, :]
    h2 = jnp.maximum(
        (a0_ref[:N, :] + a1_ref[:N, :] + g2_ref[...]) * dv
        + v2_ref[...] + b_ref[...], 0.0)
    sm = jnp.sum(h2, axis=0, keepdims=True) * (1.0 / N)
    o_ref[...] = jnp.dot(sm, cw_ref[...],
                         preferred_element_type=jnp.float32) + cb_ref[...]


def _tc_final(a0, a1, g2, dinv, v2, b, cw, cb):
    return pl.pallas_call(
        _final_body,
        out_shape=jax.ShapeDtypeStruct((1, C), jnp.float32),
    )(a0, a1, g2, dinv, v2, b, cw, cb)


def kernel(x, edge_index, W1, b1, S1w, S1b, W2, b2, S2w, S2b, Cw, Cb):
    src = edge_index[0]
    dst = edge_index[1]
    pad = jnp.full((EPAD - E,), N, dtype=edge_index.dtype)
    srcp = jnp.concatenate([src, pad])
    dstp = jnp.concatenate([dst, pad])
    iota = jnp.arange(DEG_ROWS, dtype=jnp.int32)
    zdeg = jnp.zeros((DEG_ROWS, L), jnp.float32)
    zacc = jnp.zeros((NP, H), jnp.float32)

    degp = _deg_pass(dstp, iota, zdeg)
    u = _tc_mm(x, jnp.concatenate([W1, S1w], axis=1))

    d2 = degp.reshape(NC, NP, 1)
    g1, dinv = _tc_scale(u, d2[0], d2[1])
    g1p = jnp.pad(g1, ((0, NP - N), (0, 0)))
    acc1 = _edge_pass(g1p, srcp, dstp, zacc)

    g2, v2 = _tc_layer(acc1[0], acc1[1], g1, dinv, u,
                       (b1 + S1b).reshape(1, H),
                       jnp.concatenate([W2, S2w], axis=1))
    g2p = jnp.pad(g2, ((0, NP - N), (0, 0)))
    acc2 = _edge_pass(g2p, srcp, dstp, zacc)

    return _tc_final(acc2[0], acc2[1], g2, dinv, v2,
                     (b2 + S2b).reshape(1, H), Cw, Cb.reshape(1, C))


# speedup vs baseline: 24.9666x; 1.3997x over previous
import dataclasses
import functools

import jax
import jax.numpy as jnp
from jax import lax
from jax.experimental import pallas as pl
from jax.experimental.pallas import tpu as pltpu
from jax.experimental.pallas import tpu_sc as plsc

N = 10000
E = 320000
D_IN = 128
H = 32
C = 16

NC = 2
NS = 16
L = 16
NW = NC * NS
BLK = 128
NBLK = 80
PERW = NBLK * BLK
EPAD = NW * PERW
KG = 8
NGRP = NBLK // KG
NP = 10240
DEG_ROWS = NP // L
DR_SUB = DEG_ROWS // NS
NP_SUB = NP // NS

_mesh = plsc.VectorSubcoreMesh(core_axis_name="c", subcore_axis_name="s")

_sc_params = pltpu.CompilerParams()
if "needs_layout_passes" in pltpu.CompilerParams.__dataclass_fields__:
    _sc_params = dataclasses.replace(_sc_params, needs_layout_passes=False)
if "use_tc_tiling_on_sc" in pltpu.CompilerParams.__dataclass_fields__:
    _sc_params = dataclasses.replace(_sc_params, use_tc_tiling_on_sc=False)


def _deg_body(dst_hbm, iota_hbm, z_hbm, out_hbm, dflat, hist, ibuf, deg_sh):
    c = lax.axis_index("c")
    s = lax.axis_index("s")
    w = s * NC + c
    pltpu.sync_copy(z_hbm, hist)
    pltpu.sync_copy(z_hbm.at[pl.ds(s * DR_SUB, DR_SUB)],
                    deg_sh.at[pl.ds(s * DR_SUB, DR_SUB)])
    pltpu.sync_copy(dst_hbm.at[w], dflat)
    plsc.subcore_barrier()

    ones = jnp.ones((L,), jnp.float32)

    @pl.loop(0, PERW // L)
    def _vecs(j):
        idx = dflat[pl.ds(j * L, L)]
        row = jnp.right_shift(idx, 4)
        lane = jnp.bitwise_and(idx, 15)
        plsc.addupdate_scatter(hist, [row, lane], ones)

    @pl.loop(0, DEG_ROWS // BLK)
    def _merge(k):
        pltpu.sync_copy(iota_hbm.at[pl.ds(k * BLK, BLK)], ibuf)
        pltpu.sync_copy(hist.at[pl.ds(k * BLK, BLK)], deg_sh.at[ibuf], add=True)

    plsc.subcore_barrier()
    pltpu.sync_copy(deg_sh.at[pl.ds(s * DR_SUB, DR_SUB)],
                    out_hbm.at[c, pl.ds(s * DR_SUB, DR_SUB)])


_deg_pass = pl.kernel(
    _deg_body,
    out_type=jax.ShapeDtypeStruct((NC, DEG_ROWS, L), jnp.float32),
    mesh=_mesh,
    scratch_types=[
        pltpu.VMEM((PERW,), jnp.int32),
        pltpu.VMEM((DEG_ROWS, L), jnp.float32),
        pltpu.VMEM((BLK,), jnp.int32),
        pltpu.VMEM_SHARED((DEG_ROWS, L), jnp.float32),
    ],
    compiler_params=_sc_params,
)


def _edge_body(g_hbm, src_hbm, dst_hbm, z_hbm, out_hbm,
               srcb, dstb, rows_a, rows_b, acc_sh, gsem_a, gsem_b,
               ssem_a, ssem_b):
    c = lax.axis_index("c")
    s = lax.axis_index("s")
    w = s * NC + c
    pltpu.sync_copy(z_hbm.at[pl.ds(s * NP_SUB, NP_SUB)],
                    acc_sh.at[pl.ds(s * NP_SUB, NP_SUB)])
    pltpu.sync_copy(src_hbm.at[w], srcb)
    pltpu.sync_copy(dst_hbm.at[w], dstb)
    plsc.subcore_barrier()

    def fire_gathers(grp, rows, sem):
        for j in range(KG):
            pltpu.async_copy(g_hbm.at[srcb.at[grp * KG + j]],
                             rows.at[pl.ds(j * BLK, BLK)], sem)

    def fire_scatters(grp, rows, sem):
        for j in range(KG):
            pltpu.async_copy(rows.at[pl.ds(j * BLK, BLK)],
                             acc_sh.at[dstb.at[grp * KG + j]], sem, add=True)

    def drain(ref, sem):
        pltpu.make_async_copy(z_hbm.at[pl.ds(0, KG * BLK)], ref, sem).wait()

    fire_gathers(0, rows_a, gsem_a)

    @pl.loop(0, NGRP // 2)
    def _pairs(t):
        ga = 2 * t
        gb = 2 * t + 1
        drain(rows_a, gsem_a)
        fire_gathers(gb, rows_b, gsem_b)
        fire_scatters(ga, rows_a, ssem_a)
        drain(rows_b, gsem_b)
        drain(rows_a, ssem_a)

        @pl.when(t < NGRP // 2 - 1)
        def _prefetch():
            fire_gathers(ga + 2, rows_a, gsem_a)

        fire_scatters(gb, rows_b, ssem_b)
        drain(rows_b, ssem_b)

    plsc.subcore_barrier()
    pltpu.sync_copy(acc_sh.at[pl.ds(s * NP_SUB, NP_SUB)],
                    out_hbm.at[c, pl.ds(s * NP_SUB, NP_SUB)])


_edge_pass = pl.kernel(
    _edge_body,
    out_type=jax.ShapeDtypeStruct((NC, NP, H), jnp.float32),
    mesh=_mesh,
    scratch_types=[
        pltpu.VMEM((NBLK, BLK), jnp.int32),
        pltpu.VMEM((NBLK, BLK), jnp.int32),
        pltpu.VMEM((KG * BLK, H), jnp.float32),
        pltpu.VMEM((KG * BLK, H), jnp.float32),
        pltpu.VMEM_SHARED((NP, H), jnp.float32),
        pltpu.SemaphoreType.DMA,
        pltpu.SemaphoreType.DMA,
        pltpu.SemaphoreType.DMA,
        pltpu.SemaphoreType.DMA,
    ],
    compiler_params=_sc_params,
)


def _mm_body(x_ref, w_ref, o_ref):
    o_ref[...] = jnp.dot(x_ref[...], w_ref[...],
                         preferred_element_type=jnp.float32)


def _tc_mm(x, w):
    return pl.pallas_call(
        _mm_body,
        out_shape=jax.ShapeDtypeStruct((x.shape[0], w.shape[1]), jnp.float32),
    )(x, w)


def _scale_body(u_ref, d0_ref, d1_ref, g1_ref, dinv_ref):
    deg = d0_ref[...] + d1_ref[...] + 1.0
    dinv = lax.rsqrt(deg)
    dinv_ref[...] = dinv
    g1_ref[...] = u_ref[:, :H] * dinv[:N, :]


def _tc_scale(u, d0, d1):
    return pl.pallas_call(
        _scale_body,
        out_shape=(jax.ShapeDtypeStruct((N, H), jnp.float32),
                   jax.ShapeDtypeStruct((NP, 1), jnp.float32)),
    )(u, d0, d1)


def _layer_body(a0_ref, a1_ref, g1_ref, dinv_ref, u_ref, b_ref, w2_ref,
                g2_ref, v2_ref):
    dv = dinv_ref[:N, :]
    h1 = jnp.maximum(
        (a0_ref[:N, :] + a1_ref[:N, :] + g1_ref[...]) * dv
        + u_ref[:, H:] + b_ref[...], 0.0)
    v = jnp.dot(h1, w2_ref[...], preferred_element_type=jnp.float32)
    g2_ref[...] = v[:, :H] * dv
    v2_ref[...] = v[:, H:]


def _tc_layer(a0, a1, g1, dinv, u, b, w2):
    return pl.pallas_call(
        _layer_body,
        out_shape=(jax.ShapeDtypeStruct((N, H), jnp.float32),
                   jax.ShapeDtypeStruct((N, H), jnp.float32)),
    )(a0, a1, g1, dinv, u, b, w2)


def _final_body(a0_ref, a1_ref, g2_ref, dinv_ref, v2_ref, b_ref, cw_ref,
                cb_ref, o_ref):
    dv = dinv_ref[:N, :]
    h2 = jnp.maximum(
        (a0_ref[:N, :] + a1_ref[:N, :] + g2_ref[...]) * dv
        + v2_ref[...] + b_ref[...], 0.0)
    sm = jnp.sum(h2, axis=0, keepdims=True) * (1.0 / N)
    o_ref[...] = jnp.dot(sm, cw_ref[...],
                         preferred_element_type=jnp.float32) + cb_ref[...]


def _tc_final(a0, a1, g2, dinv, v2, b, cw, cb):
    return pl.pallas_call(
        _final_body,
        out_shape=jax.ShapeDtypeStruct((1, C), jnp.float32),
    )(a0, a1, g2, dinv, v2, b, cw, cb)


def kernel(x, edge_index, W1, b1, S1w, S1b, W2, b2, S2w, S2b, Cw, Cb):
    src = edge_index[0]
    dst = edge_index[1]
    pad = jnp.full((EPAD - E,), N, dtype=edge_index.dtype)
    srcp = jnp.concatenate([src, pad]).reshape(NW, NBLK, BLK)
    dstp = jnp.concatenate([dst, pad]).reshape(NW, NBLK, BLK)
    dstf = dstp.reshape(NW, PERW)
    iota = jnp.arange(DEG_ROWS, dtype=jnp.int32)
    zdeg = jnp.zeros((DEG_ROWS, L), jnp.float32)
    zacc = jnp.zeros((NP, H), jnp.float32)

    degp = _deg_pass(dstf, iota, zdeg)
    u = _tc_mm(x, jnp.concatenate([W1, S1w], axis=1))

    d2 = degp.reshape(NC, NP, 1)
    g1, dinv = _tc_scale(u, d2[0], d2[1])
    g1p = jnp.pad(g1, ((0, NP - N), (0, 0)))
    acc1 = _edge_pass(g1p, srcp, dstp, zacc)

    g2, v2 = _tc_layer(acc1[0], acc1[1], g1, dinv, u,
                       (b1 + S1b).reshape(1, H),
                       jnp.concatenate([W2, S2w], axis=1))
    g2p = jnp.pad(g2, ((0, NP - N), (0, 0)))
    acc2 = _edge_pass(g2p, srcp, dstp, zacc)

    return _tc_final(acc2[0], acc2[1], g2, dinv, v2,
                     (b2 + S2b).reshape(1, H), Cw, Cb.reshape(1, C))


# speedup vs baseline: 27.2860x; 1.0929x over previous
import dataclasses
import functools

import jax
import jax.numpy as jnp
from jax import lax
from jax.experimental import pallas as pl
from jax.experimental.pallas import tpu as pltpu
from jax.experimental.pallas import tpu_sc as plsc

N = 10000
E = 320000
D_IN = 128
H = 32
C = 16

NC = 2
NS = 16
L = 16
NW = NC * NS
BLK = 128
NBLK = 80
PERW = NBLK * BLK
EPAD = NW * PERW
KG = 8
NGRP = NBLK // KG
NP = 10240
DEG_ROWS = NP // L
DR_SUB = DEG_ROWS // NS
NP_SUB = NP // NS

_mesh = plsc.VectorSubcoreMesh(core_axis_name="c", subcore_axis_name="s")

_sc_params = pltpu.CompilerParams()
if "needs_layout_passes" in pltpu.CompilerParams.__dataclass_fields__:
    _sc_params = dataclasses.replace(_sc_params, needs_layout_passes=False)
if "use_tc_tiling_on_sc" in pltpu.CompilerParams.__dataclass_fields__:
    _sc_params = dataclasses.replace(_sc_params, use_tc_tiling_on_sc=False)


def _deg_body(dst_hbm, iota_hbm, z_hbm, out_hbm, dflat, hist, ibuf, deg_sh):
    c = lax.axis_index("c")
    s = lax.axis_index("s")
    w = s * NC + c
    pltpu.sync_copy(z_hbm, hist)
    pltpu.sync_copy(z_hbm.at[pl.ds(s * DR_SUB, DR_SUB)],
                    deg_sh.at[pl.ds(s * DR_SUB, DR_SUB)])
    pltpu.sync_copy(dst_hbm.at[w], dflat)
    plsc.subcore_barrier()

    ones = jnp.ones((L,), jnp.float32)

    @pl.loop(0, PERW // L)
    def _vecs(j):
        idx = dflat[pl.ds(j * L, L)]
        row = jnp.right_shift(idx, 4)
        lane = jnp.bitwise_and(idx, 15)
        plsc.addupdate_scatter(hist, [row, lane], ones)

    @pl.loop(0, DEG_ROWS // BLK)
    def _merge(k):
        pltpu.sync_copy(iota_hbm.at[pl.ds(k * BLK, BLK)], ibuf)
        pltpu.sync_copy(hist.at[pl.ds(k * BLK, BLK)], deg_sh.at[ibuf], add=True)

    plsc.subcore_barrier()
    pltpu.sync_copy(deg_sh.at[pl.ds(s * DR_SUB, DR_SUB)],
                    out_hbm.at[c, pl.ds(s * DR_SUB, DR_SUB)])


_deg_pass = pl.kernel(
    _deg_body,
    out_type=jax.ShapeDtypeStruct((NC, DEG_ROWS, L), jnp.float32),
    mesh=_mesh,
    scratch_types=[
        pltpu.VMEM((PERW,), jnp.int32),
        pltpu.VMEM((DEG_ROWS, L), jnp.float32),
        pltpu.VMEM((BLK,), jnp.int32),
        pltpu.VMEM_SHARED((DEG_ROWS, L), jnp.float32),
    ],
    compiler_params=_sc_params,
)


def _edge_body(g_hbm, src_hbm, dst_hbm, z_hbm, out_hbm,
               srcb, dstb, rows_a, rows_b, acc_sh, gsem_a, gsem_b,
               ssem_a, ssem_b):
    c = lax.axis_index("c")
    s = lax.axis_index("s")
    w = s * NC + c
    pltpu.sync_copy(z_hbm.at[pl.ds(s * NP_SUB, NP_SUB)],
                    acc_sh.at[pl.ds(s * NP_SUB, NP_SUB)])
    pltpu.sync_copy(src_hbm.at[w], srcb)
    pltpu.sync_copy(dst_hbm.at[w], dstb)
    plsc.subcore_barrier()

    def fire_gathers(grp, rows, sem):
        for j in range(KG):
            pltpu.async_copy(g_hbm.at[srcb.at[grp * KG + j]],
                             rows.at[pl.ds(j * BLK, BLK)], sem)

    def fire_scatters(grp, rows, sem):
        for j in range(KG):
            pltpu.async_copy(rows.at[pl.ds(j * BLK, BLK)],
                             acc_sh.at[dstb.at[grp * KG + j]], sem, add=True)

    def drain(ref, sem):
        pltpu.make_async_copy(z_hbm.at[pl.ds(0, KG * BLK)], ref, sem).wait()

    fire_gathers(0, rows_a, gsem_a)

    @pl.loop(0, NGRP // 2)
    def _pairs(t):
        ga = 2 * t
        gb = 2 * t + 1
        drain(rows_a, gsem_a)
        fire_gathers(gb, rows_b, gsem_b)
        fire_scatters(ga, rows_a, ssem_a)
        drain(rows_b, gsem_b)
        drain(rows_a, ssem_a)

        @pl.when(t < NGRP // 2 - 1)
        def _prefetch():
            fire_gathers(ga + 2, rows_a, gsem_a)

        fire_scatters(gb, rows_b, ssem_b)
        drain(rows_b, ssem_b)

    plsc.subcore_barrier()
    pltpu.sync_copy(acc_sh.at[pl.ds(s * NP_SUB, NP_SUB)],
                    out_hbm.at[c, pl.ds(s * NP_SUB, NP_SUB)])


_edge_pass = pl.kernel(
    _edge_body,
    out_type=jax.ShapeDtypeStruct((NC, NP, H), jnp.float32),
    mesh=_mesh,
    scratch_types=[
        pltpu.VMEM((NBLK, BLK), jnp.int32),
        pltpu.VMEM((NBLK, BLK), jnp.int32),
        pltpu.VMEM((KG * BLK, H), jnp.float32),
        pltpu.VMEM((KG * BLK, H), jnp.float32),
        pltpu.VMEM_SHARED((NP, H), jnp.float32),
        pltpu.SemaphoreType.DMA,
        pltpu.SemaphoreType.DMA,
        pltpu.SemaphoreType.DMA,
        pltpu.SemaphoreType.DMA,
    ],
    compiler_params=_sc_params,
)


def _mm_body(x_ref, w_ref, o_ref):
    o_ref[...] = jnp.dot(x_ref[...], w_ref[...],
                         preferred_element_type=jnp.float32)


def _tc_mm(x, w):
    return pl.pallas_call(
        _mm_body,
        out_shape=jax.ShapeDtypeStruct((x.shape[0], w.shape[1]), jnp.float32),
    )(x, w)


def _scale_body(u_ref, d0_ref, d1_ref, g1_ref, dinv_ref):
    deg = d0_ref[...] + d1_ref[...] + 1.0
    dinv = lax.rsqrt(deg)
    dinv_ref[...] = dinv
    g1_ref[...] = u_ref[:, :H] * dinv[:N, :]


def _tc_scale(u, d0, d1):
    return pl.pallas_call(
        _scale_body,
        out_shape=(jax.ShapeDtypeStruct((N, H), jnp.float32),
                   jax.ShapeDtypeStruct((NP, 1), jnp.float32)),
    )(u, d0, d1)


def _layer_body(a0_ref, a1_ref, g1_ref, dinv_ref, u_ref, b_ref, w2_ref,
                g2_ref, v2_ref):
    dv = dinv_ref[:N, :]
    h1 = jnp.maximum(
        (a0_ref[:N, :] + a1_ref[:N, :] + g1_ref[...]) * dv
        + u_ref[:, H:] + b_ref[...], 0.0)
    v = jnp.dot(h1, w2_ref[...], preferred_element_type=jnp.float32)
    g2_ref[...] = v[:, :H] * dv
    v2_ref[...] = v[:, H:]


def _tc_layer(a0, a1, g1, dinv, u, b, w2):
    return pl.pallas_call(
        _layer_body,
        out_shape=(jax.ShapeDtypeStruct((N, H), jnp.float32),
                   jax.ShapeDtypeStruct((N, H), jnp.float32)),
    )(a0, a1, g1, dinv, u, b, w2)


def _final_body(a0_ref, a1_ref, g2_ref, dinv_ref, v2_ref, b_ref, cw_ref,
                cb_ref, o_ref):
    dv = dinv_ref[:N, :]
    h2 = jnp.maximum(
        (a0_ref[:N, :] + a1_ref[:N, :] + g2_ref[...]) * dv
        + v2_ref[...] + b_ref[...], 0.0)
    sm = jnp.sum(h2, axis=0, keepdims=True) * (1.0 / N)
    o_ref[...] = jnp.dot(sm, cw_ref[...],
                         preferred_element_type=jnp.float32) + cb_ref[...]


def _tc_final(a0, a1, g2, dinv, v2, b, cw, cb):
    return pl.pallas_call(
        _final_body,
        out_shape=jax.ShapeDtypeStruct((1, C), jnp.float32),
    )(a0, a1, g2, dinv, v2, b, cw, cb)


def kernel(x, edge_index, W1, b1, S1w, S1b, W2, b2, S2w, S2b, Cw, Cb):
    src = edge_index[0]
    dst = edge_index[1]
    npad = PERW - E // NW
    pad_src = jnp.full((NW, npad), N, dtype=edge_index.dtype)
    pad_dst = jnp.broadcast_to(N + jnp.arange(npad, dtype=edge_index.dtype),
                               (NW, npad))
    srcp = jnp.concatenate([src.reshape(NW, E // NW), pad_src],
                           axis=1).reshape(NW, NBLK, BLK)
    dstp = jnp.concatenate([dst.reshape(NW, E // NW), pad_dst],
                           axis=1).reshape(NW, NBLK, BLK)
    dstf = dstp.reshape(NW, PERW)
    iota = jnp.arange(DEG_ROWS, dtype=jnp.int32)
    zdeg = jnp.zeros((DEG_ROWS, L), jnp.float32)
    zacc = jnp.zeros((NP, H), jnp.float32)

    degp = _deg_pass(dstf, iota, zdeg)
    u = _tc_mm(x, jnp.concatenate([W1, S1w], axis=1))

    d2 = degp.reshape(NC, NP, 1)
    g1, dinv = _tc_scale(u, d2[0], d2[1])
    g1p = jnp.pad(g1, ((0, NP - N), (0, 0)))
    acc1 = _edge_pass(g1p, srcp, dstp, zacc)

    g2, v2 = _tc_layer(acc1[0], acc1[1], g1, dinv, u,
                       (b1 + S1b).reshape(1, H),
                       jnp.concatenate([W2, S2w], axis=1))
    g2p = jnp.pad(g2, ((0, NP - N), (0, 0)))
    acc2 = _edge_pass(g2p, srcp, dstp, zacc)

    return _tc_final(acc2[0], acc2[1], g2, dinv, v2,
                     (b2 + S2b).reshape(1, H), Cw, Cb.reshape(1, C))


# speedup vs baseline: 44.4590x; 1.6294x over previous
import dataclasses
import functools

import jax
import jax.numpy as jnp
from jax import lax
from jax.experimental import pallas as pl
from jax.experimental.pallas import tpu as pltpu
from jax.experimental.pallas import tpu_sc as plsc

N = 10000
E = 320000
D_IN = 128
H = 32
C = 16

NC = 2
NS = 16
L = 16
NW = NC * NS
BLK = 128
NBLK = 80
PERW = NBLK * BLK
EPAD = NW * PERW
KG = 8
NGRP = NBLK // KG
NP = 10240
DEG_ROWS = NP // L
DR_SUB = DEG_ROWS // NS
NP_SUB = NP // NS

_mesh = plsc.VectorSubcoreMesh(core_axis_name="c", subcore_axis_name="s")

_sc_params = pltpu.CompilerParams()
if "needs_layout_passes" in pltpu.CompilerParams.__dataclass_fields__:
    _sc_params = dataclasses.replace(_sc_params, needs_layout_passes=False)
if "use_tc_tiling_on_sc" in pltpu.CompilerParams.__dataclass_fields__:
    _sc_params = dataclasses.replace(_sc_params, use_tc_tiling_on_sc=False)


def _deg_body(dst_hbm, iota_hbm, z_hbm, out_hbm, dflat, hist, ibuf, deg_sh):
    c = lax.axis_index("c")
    s = lax.axis_index("s")
    w = s * NC + c
    pltpu.sync_copy(z_hbm, hist)
    pltpu.sync_copy(z_hbm.at[pl.ds(s * DR_SUB, DR_SUB)],
                    deg_sh.at[pl.ds(s * DR_SUB, DR_SUB)])
    pltpu.sync_copy(dst_hbm.at[w], dflat)
    plsc.subcore_barrier()

    ones = jnp.ones((L,), jnp.float32)

    @pl.loop(0, PERW // L)
    def _vecs(j):
        idx = dflat[pl.ds(j * L, L)]
        row = jnp.right_shift(idx, 4)
        lane = jnp.bitwise_and(idx, 15)
        plsc.addupdate_scatter(hist, [row, lane], ones)

    @pl.loop(0, DEG_ROWS // BLK)
    def _merge(k):
        pltpu.sync_copy(iota_hbm.at[pl.ds(k * BLK, BLK)], ibuf)
        pltpu.sync_copy(hist.at[pl.ds(k * BLK, BLK)], deg_sh.at[ibuf], add=True)

    plsc.subcore_barrier()
    pltpu.sync_copy(deg_sh.at[pl.ds(s * DR_SUB, DR_SUB)],
                    out_hbm.at[c, pl.ds(s * DR_SUB, DR_SUB)])


_deg_pass = pl.kernel(
    _deg_body,
    out_type=jax.ShapeDtypeStruct((NC, DEG_ROWS, L), jnp.float32),
    mesh=_mesh,
    scratch_types=[
        pltpu.VMEM((PERW,), jnp.int32),
        pltpu.VMEM((DEG_ROWS, L), jnp.float32),
        pltpu.VMEM((BLK,), jnp.int32),
        pltpu.VMEM_SHARED((DEG_ROWS, L), jnp.float32),
    ],
    compiler_params=_sc_params,
)


def _edge_body(g_hbm, src_hbm, dst_hbm, z_hbm, out_hbm,
               srcb, dstb, rows_a, rows_b, g_sh, acc_sh, gsem_a, gsem_b,
               ssem_a, ssem_b):
    c = lax.axis_index("c")
    s = lax.axis_index("s")
    w = s * NC + c
    pltpu.sync_copy(z_hbm.at[pl.ds(s * NP_SUB, NP_SUB)],
                    acc_sh.at[pl.ds(s * NP_SUB, NP_SUB)])
    pltpu.sync_copy(g_hbm.at[pl.ds(s * NP_SUB, NP_SUB)],
                    g_sh.at[pl.ds(s * NP_SUB, NP_SUB)])
    pltpu.sync_copy(src_hbm.at[w], srcb)
    pltpu.sync_copy(dst_hbm.at[w], dstb)
    plsc.subcore_barrier()

    def fire_gathers(grp, rows, sem):
        for j in range(KG):
            pltpu.async_copy(g_sh.at[srcb.at[grp * KG + j]],
                             rows.at[pl.ds(j * BLK, BLK)], sem)

    def fire_scatters(grp, rows, sem):
        for j in range(KG):
            pltpu.async_copy(rows.at[pl.ds(j * BLK, BLK)],
                             acc_sh.at[dstb.at[grp * KG + j]], sem, add=True)

    def drain(ref, sem):
        pltpu.make_async_copy(z_hbm.at[pl.ds(0, KG * BLK)], ref, sem).wait()

    fire_gathers(0, rows_a, gsem_a)

    @pl.loop(0, NGRP // 2)
    def _pairs(t):
        ga = 2 * t
        gb = 2 * t + 1
        drain(rows_a, gsem_a)
        fire_gathers(gb, rows_b, gsem_b)
        fire_scatters(ga, rows_a, ssem_a)
        drain(rows_b, gsem_b)
        drain(rows_a, ssem_a)

        @pl.when(t < NGRP // 2 - 1)
        def _prefetch():
            fire_gathers(ga + 2, rows_a, gsem_a)

        fire_scatters(gb, rows_b, ssem_b)
        drain(rows_b, ssem_b)

    plsc.subcore_barrier()
    pltpu.sync_copy(acc_sh.at[pl.ds(s * NP_SUB, NP_SUB)],
                    out_hbm.at[c, pl.ds(s * NP_SUB, NP_SUB)])


_edge_pass = pl.kernel(
    _edge_body,
    out_type=jax.ShapeDtypeStruct((NC, NP, H), jnp.float32),
    mesh=_mesh,
    scratch_types=[
        pltpu.VMEM((NBLK, BLK), jnp.int32),
        pltpu.VMEM((NBLK, BLK), jnp.int32),
        pltpu.VMEM((KG * BLK, H), jnp.float32),
        pltpu.VMEM((KG * BLK, H), jnp.float32),
        pltpu.VMEM_SHARED((NP, H), jnp.float32),
        pltpu.VMEM_SHARED((NP, H), jnp.float32),
        pltpu.SemaphoreType.DMA,
        pltpu.SemaphoreType.DMA,
        pltpu.SemaphoreType.DMA,
        pltpu.SemaphoreType.DMA,
    ],
    compiler_params=_sc_params,
)


def _mm_body(x_ref, w_ref, o_ref):
    o_ref[...] = jnp.dot(x_ref[...], w_ref[...],
                         preferred_element_type=jnp.float32)


def _tc_mm(x, w):
    return pl.pallas_call(
        _mm_body,
        out_shape=jax.ShapeDtypeStruct((x.shape[0], w.shape[1]), jnp.float32),
    )(x, w)


def _scale_body(u_ref, d0_ref, d1_ref, g1_ref, dinv_ref):
    deg = d0_ref[...] + d1_ref[...] + 1.0
    dinv = lax.rsqrt(deg)
    dinv_ref[...] = dinv
    g1_ref[...] = u_ref[:, :H] * dinv[:N, :]


def _tc_scale(u, d0, d1):
    return pl.pallas_call(
        _scale_body,
        out_shape=(jax.ShapeDtypeStruct((N, H), jnp.float32),
                   jax.ShapeDtypeStruct((NP, 1), jnp.float32)),
    )(u, d0, d1)


def _layer_body(a0_ref, a1_ref, g1_ref, dinv_ref, u_ref, b_ref, w2_ref,
                g2_ref, v2_ref):
    dv = dinv_ref[:N, :]
    h1 = jnp.maximum(
        (a0_ref[:N, :] + a1_ref[:N, :] + g1_ref[...]) * dv
        + u_ref[:, H:] + b_ref[...], 0.0)
    v = jnp.dot(h1, w2_ref[...], preferred_element_type=jnp.float32)
    g2_ref[...] = v[:, :H] * dv
    v2_ref[...] = v[:, H:]


def _tc_layer(a0, a1, g1, dinv, u, b, w2):
    return pl.pallas_call(
        _layer_body,
        out_shape=(jax.ShapeDtypeStruct((N, H), jnp.float32),
                   jax.ShapeDtypeStruct((N, H), jnp.float32)),
    )(a0, a1, g1, dinv, u, b, w2)


def _final_body(a0_ref, a1_ref, g2_ref, dinv_ref, v2_ref, b_ref, cw_ref,
                cb_ref, o_ref):
    dv = dinv_ref[:N, :]
    h2 = jnp.maximum(
        (a0_ref[:N, :] + a1_ref[:N, :] + g2_ref[...]) * dv
        + v2_ref[...] + b_ref[...], 0.0)
    sm = jnp.sum(h2, axis=0, keepdims=True) * (1.0 / N)
    o_ref[...] = jnp.dot(sm, cw_ref[...],
                         preferred_element_type=jnp.float32) + cb_ref[...]


def _tc_final(a0, a1, g2, dinv, v2, b, cw, cb):
    return pl.pallas_call(
        _final_body,
        out_shape=jax.ShapeDtypeStruct((1, C), jnp.float32),
    )(a0, a1, g2, dinv, v2, b, cw, cb)


def kernel(x, edge_index, W1, b1, S1w, S1b, W2, b2, S2w, S2b, Cw, Cb):
    src = edge_index[0]
    dst = edge_index[1]
    npad = PERW - E // NW
    pad_src = jnp.full((NW, npad), N, dtype=edge_index.dtype)
    pad_dst = jnp.broadcast_to(N + jnp.arange(npad, dtype=edge_index.dtype),
                               (NW, npad))
    srcp = jnp.concatenate([src.reshape(NW, E // NW), pad_src],
                           axis=1).reshape(NW, NBLK, BLK)
    dstp = jnp.concatenate([dst.reshape(NW, E // NW), pad_dst],
                           axis=1).reshape(NW, NBLK, BLK)
    dstf = dstp.reshape(NW, PERW)
    iota = jnp.arange(DEG_ROWS, dtype=jnp.int32)
    zdeg = jnp.zeros((DEG_ROWS, L), jnp.float32)
    zacc = jnp.zeros((NP, H), jnp.float32)

    degp = _deg_pass(dstf, iota, zdeg)
    u = _tc_mm(x, jnp.concatenate([W1, S1w], axis=1))

    d2 = degp.reshape(NC, NP, 1)
    g1, dinv = _tc_scale(u, d2[0], d2[1])
    g1p = jnp.pad(g1, ((0, NP - N), (0, 0)))
    acc1 = _edge_pass(g1p, srcp, dstp, zacc)

    g2, v2 = _tc_layer(acc1[0], acc1[1], g1, dinv, u,
                       (b1 + S1b).reshape(1, H),
                       jnp.concatenate([W2, S2w], axis=1))
    g2p = jnp.pad(g2, ((0, NP - N), (0, 0)))
    acc2 = _edge_pass(g2p, srcp, dstp, zacc)

    return _tc_final(acc2[0], acc2[1], g2, dinv, v2,
                     (b2 + S2b).reshape(1, H), Cw, Cb.reshape(1, C))
